# baseline (speedup 1.0000x reference)
import jax
import jax.numpy as jnp
from jax import lax
from jax.experimental import pallas as pl
from jax.experimental.pallas import tpu as pltpu
from jax.experimental.pallas import tpu_sc as plsc

_B, _X, _Y, _Z, _C = 2, 96, 96, 96, 4
_XYZ = _X * _Y * _Z
_N = _B * _XYZ
_DIM2 = _Y * _Z
_DIM3 = _Z
_NW = 32
_PT = _N // _NW
_K = 1024
_NSUB = _K // 128
_NCH = _PT // _K
_OFFS = (0, 1, _DIM3, _DIM3 + 1, _DIM2, _DIM2 + 1, _DIM2 + _DIM3,
         _DIM2 + _DIM3 + 1)


def _tri_body(im_hbm, xs_hbm, ys_hbm, zs_hbm, out_hbm,
              xv, yv, zv, idx, rows, outv, sem):
    wid = lax.axis_index("s") * 2 + lax.axis_index("c")
    tile_base = wid * _PT
    bb = jnp.where(tile_base >= _XYZ, _XYZ, 0).astype(jnp.int32)

    def chunk_body(g, carry):
        base = tile_base + g * _K
        pltpu.sync_copy(xs_hbm.at[pl.ds(base, _K)], xv)
        pltpu.sync_copy(ys_hbm.at[pl.ds(base, _K)], yv)
        pltpu.sync_copy(zs_hbm.at[pl.ds(base, _K)], zv)

        def idx_body(i, c2):
            s = pl.ds(i * 16, 16)
            x = xv[s]
            y = yv[s]
            z = zv[s]
            xi = jnp.minimum(x.astype(jnp.int32), _X - 2)
            yi = jnp.minimum(y.astype(jnp.int32), _Y - 2)
            zi = jnp.minimum(z.astype(jnp.int32), _Z - 2)
            xv[s] = x - xi.astype(jnp.float32)
            yv[s] = y - yi.astype(jnp.float32)
            zv[s] = z - zi.astype(jnp.float32)
            i0 = xi * _DIM2 + yi * _DIM3 + zi + bb
            j = i >> 3
            off = (i & 7) * 16
            for c in range(8):
                idx[c, j, pl.ds(off, 16)] = i0 + _OFFS[c]
            return c2

        lax.fori_loop(0, _K // 16, idx_body, 0)

        copies = []
        for c in range(8):
            for jj in range(_NSUB):
                copies.append(
                    pltpu.async_copy(im_hbm.at[idx.at[c, jj]],
                                     rows.at[c, jj], sem))
        for cp in copies:
            cp.wait()

        lane = lax.iota(jnp.int32, 16)
        pt_in_reg = lane >> 2
        ch_idx = lane & 3

        def blend_body(r, c2):
            p0 = r * 4
            jj = p0 >> 7
            pt_idx = pt_in_reg + (p0 & 127)
            ptc = pt_in_reg + p0
            xd = plsc.load_gather(xv, [ptc])
            yd = plsc.load_gather(yv, [ptc])
            zd = plsc.load_gather(zv, [ptc])
            jv = jnp.full((16,), jj, jnp.int32)
            vals = []
            for c in range(8):
                cv = jnp.full((16,), c, jnp.int32)
                vals.append(plsc.load_gather(rows, [cv, jv, pt_idx, ch_idx]))
            ia, ib, ic, id_, ie, if_, ig, ih = vals
            mx = 1.0 - xd
            my = 1.0 - yd
            mz = 1.0 - zd
            cae = ia * mx + ie * xd
            cbf = ib * mx + if_ * xd
            ccg = ic * mx + ig * xd
            cdh = id_ * mx + ih * xd
            caecg = cae * my + ccg * yd
            cbfdh = cbf * my + cdh * yd
            outv[pl.ds(r * 16, 16)] = caecg * mz + cbfdh * zd
            return c2

        lax.fori_loop(0, _K * _C // 16, blend_body, 0)

        pltpu.sync_copy(outv, out_hbm.at[pl.ds(base * _C, _K * _C)])
        return carry

    lax.fori_loop(0, _NCH, chunk_body, 0)


_tri = pl.kernel(
    _tri_body,
    mesh=plsc.VectorSubcoreMesh(core_axis_name="c", subcore_axis_name="s"),
    out_type=jax.ShapeDtypeStruct((_N * _C,), jnp.float32),
    compiler_params=pltpu.CompilerParams(
        needs_layout_passes=False, use_tc_tiling_on_sc=False),
    scratch_types=[
        pltpu.VMEM((_K,), jnp.float32),
        pltpu.VMEM((_K,), jnp.float32),
        pltpu.VMEM((_K,), jnp.float32),
        pltpu.VMEM((8, _NSUB, 128), jnp.int32),
        pltpu.VMEM((8, _NSUB, 128, _C), jnp.float32),
        pltpu.VMEM((_K * _C,), jnp.float32),
        pltpu.SemaphoreType.DMA,
    ],
)


@jax.jit
def kernel(im, defgrid):
    B, X, Y, Z, C = im.shape
    im_flat = im.reshape(-1, C)
    g = defgrid.reshape(-1, 3)
    out = _tri(im_flat, g[:, 0], g[:, 1], g[:, 2])
    return out.reshape(B, X, Y, Z, C)
